# Initial kernel scaffold; baseline (speedup 1.0000x reference)
import jax
import jax.numpy as jnp
from jax.experimental import pallas as pl


def kernel(x, tok_table, pos_table, gamma, beta):
    raise NotImplementedError("write your pallas kernel here")


# speedup vs baseline: 1.3240x; 1.3240x over previous
import functools

import jax
import jax.numpy as jnp
from jax import lax
from jax.experimental import pallas as pl
from jax.experimental.pallas import tpu as pltpu
from jax.experimental.pallas import tpu_sc as plsc

D_MODEL = 64
MAXLEN = 200
LANES = 16
NUM_WORKERS = 32
CHUNK = 512
SUB = 64
EPS = 1e-5


def _lane_sum(v):
    idx = lax.iota(jnp.int32, LANES)
    for k in (8, 4, 2, 1):
        v = v + v.at[idx ^ k].get(mode="promise_in_bounds")
    return v


def _rsqrt_vec(v):
    i = lax.bitcast_convert_type(v, jnp.int32)
    i = jnp.int32(0x5F3759DF) - (i >> 1)
    y = lax.bitcast_convert_type(i, jnp.float32)
    for _ in range(4):
        y = y * (1.5 - 0.5 * v * y * y)
    return y


def _make_sc_kernel(n_rows):
    rows_per_w = n_rows // NUM_WORKERS
    n_chunks = rows_per_w // CHUNK
    mesh = plsc.VectorSubcoreMesh(core_axis_name="c", subcore_axis_name="s")

    @functools.partial(
        pl.kernel,
        mesh=mesh,
        compiler_params=pltpu.CompilerParams(use_tc_tiling_on_sc=False),
        out_type=jax.ShapeDtypeStruct((n_rows, D_MODEL), jnp.float32),
        scratch_types=[
            pltpu.VMEM((CHUNK // SUB, SUB), jnp.int32),
            pltpu.VMEM((CHUNK, D_MODEL), jnp.float32),
            pltpu.VMEM((MAXLEN, D_MODEL), jnp.float32),
            pltpu.VMEM((2, D_MODEL), jnp.float32),
            pltpu.SemaphoreType.DMA,
        ],
    )
    def sc_embed(x_hbm, tok_hbm, pos_hbm, gam_hbm, bet_hbm, out_hbm,
                 idx_v, rows_v, pos_v, gb_v, sem):
        cid = lax.axis_index("c")
        sid = lax.axis_index("s")
        wid = sid * 2 + cid
        base = wid * rows_per_w

        pltpu.sync_copy(pos_hbm, pos_v)
        pltpu.sync_copy(gam_hbm, gb_v.at[0])
        pltpu.sync_copy(bet_hbm, gb_v.at[1])

        g = [gb_v[0, pl.ds(LANES * j, LANES)] for j in range(4)]
        b = [gb_v[1, pl.ds(LANES * j, LANES)] for j in range(4)]

        def chunk_fn(i, carry):
            row0 = base + i * CHUNK
            jbase = pl.multiple_of(row0 // SUB, 8)
            pltpu.sync_copy(x_hbm.at[pl.ds(jbase, CHUNK // SUB)], idx_v)
            copies = [
                pltpu.async_copy(tok_hbm.at[idx_v.at[j]],
                                 rows_v.at[pl.ds(j * SUB, SUB)], sem)
                for j in range(CHUNK // SUB)
            ]
            for c in copies:
                c.wait()

            def row_fn(r, carry2):
                p = lax.rem(row0 + r, MAXLEN)
                t = [rows_v[r, pl.ds(LANES * j, LANES)]
                     + pos_v[p, pl.ds(LANES * j, LANES)]
                     for j in range(4)]
                s = (t[0] + t[1]) + (t[2] + t[3])
                mean = _lane_sum(s) * (1.0 / D_MODEL)
                c0 = [tj - mean for tj in t]
                q = (c0[0] * c0[0] + c0[1] * c0[1]) + (c0[2] * c0[2] + c0[3] * c0[3])
                var = _lane_sum(q) * (1.0 / D_MODEL)
                rstd = _rsqrt_vec(var + EPS)
                for j in range(4):
                    rows_v[r, pl.ds(LANES * j, LANES)] = c0[j] * rstd * g[j] + b[j]
                return carry2

            lax.fori_loop(0, CHUNK, row_fn, 0)
            pltpu.sync_copy(rows_v, out_hbm.at[pl.ds(row0, CHUNK)])
            return carry

        lax.fori_loop(0, n_chunks, chunk_fn, 0)

    return sc_embed


def kernel(x, tok_table, pos_table, gamma, beta):
    bsz, seq = x.shape
    n_rows = bsz * seq
    assert n_rows % (NUM_WORKERS * CHUNK) == 0
    assert seq == MAXLEN and tok_table.shape[1] == D_MODEL
    x_flat = x.reshape(n_rows // SUB, SUB).astype(jnp.int32)
    sc = _make_sc_kernel(n_rows)
    out = sc(x_flat, tok_table, pos_table, gamma, beta)
    return out.reshape(bsz, seq, D_MODEL)


# speedup vs baseline: 2.2724x; 1.7163x over previous
import functools

import jax
import jax.numpy as jnp
from jax import lax
from jax.experimental import pallas as pl
from jax.experimental.pallas import tpu as pltpu
from jax.experimental.pallas import tpu_sc as plsc

D_MODEL = 64
MAXLEN = 200
LANES = 16
NUM_WORKERS = 32
CHUNK = 512
SUB = 64
EPS = 1e-5


def _lane_sum(v):
    idx = lax.iota(jnp.int32, LANES)
    for k in (8, 4, 2, 1):
        v = v + v.at[idx ^ k].get(mode="promise_in_bounds")
    return v


def _rsqrt_vec(v):
    i = lax.bitcast_convert_type(v, jnp.int32)
    i = jnp.int32(0x5F3759DF) - (i >> 1)
    y = lax.bitcast_convert_type(i, jnp.float32)
    for _ in range(4):
        y = y * (1.5 - 0.5 * v * y * y)
    return y


def _make_sc_kernel(n_rows):
    rows_per_w = n_rows // NUM_WORKERS
    n_chunks = rows_per_w // CHUNK
    mesh = plsc.VectorSubcoreMesh(core_axis_name="c", subcore_axis_name="s")

    @functools.partial(
        pl.kernel,
        mesh=mesh,
        compiler_params=pltpu.CompilerParams(use_tc_tiling_on_sc=False),
        out_type=jax.ShapeDtypeStruct((n_rows, D_MODEL), jnp.float32),
        scratch_types=[
            pltpu.VMEM((CHUNK // SUB, SUB), jnp.int32),
            pltpu.VMEM((CHUNK, D_MODEL), jnp.float32),
            pltpu.VMEM((MAXLEN, D_MODEL), jnp.float32),
            pltpu.VMEM((2, D_MODEL), jnp.float32),
            pltpu.SemaphoreType.DMA,
        ],
    )
    def sc_embed(x_hbm, tok_hbm, pos_hbm, gam_hbm, bet_hbm, out_hbm,
                 idx_v, rows_v, pos_v, gb_v, sem):
        cid = lax.axis_index("c")
        sid = lax.axis_index("s")
        wid = sid * 2 + cid
        base = wid * rows_per_w

        pltpu.sync_copy(pos_hbm, pos_v)
        pltpu.sync_copy(gam_hbm, gb_v.at[0])
        pltpu.sync_copy(bet_hbm, gb_v.at[1])

        g = [gb_v[0, pl.ds(LANES * j, LANES)] for j in range(4)]
        b = [gb_v[1, pl.ds(LANES * j, LANES)] for j in range(4)]

        def chunk_fn(i, carry):
            row0 = base + i * CHUNK
            jbase = pl.multiple_of(row0 // SUB, 8)
            pltpu.sync_copy(x_hbm.at[pl.ds(jbase, CHUNK // SUB)], idx_v)
            copies = [
                pltpu.async_copy(tok_hbm.at[idx_v.at[j]],
                                 rows_v.at[pl.ds(j * SUB, SUB)], sem)
                for j in range(CHUNK // SUB)
            ]
            for c in copies:
                c.wait()

            @plsc.parallel_loop(0, CHUNK, unroll=4)
            def row_fn(r):
                p = lax.rem(row0 + r, MAXLEN)
                t = [rows_v[r, pl.ds(LANES * j, LANES)]
                     + pos_v[p, pl.ds(LANES * j, LANES)]
                     for j in range(4)]
                s = (t[0] + t[1]) + (t[2] + t[3])
                mean = _lane_sum(s) * (1.0 / D_MODEL)
                c0 = [tj - mean for tj in t]
                q = (c0[0] * c0[0] + c0[1] * c0[1]) + (c0[2] * c0[2] + c0[3] * c0[3])
                var = _lane_sum(q) * (1.0 / D_MODEL)
                rstd = _rsqrt_vec(var + EPS)
                for j in range(4):
                    rows_v[r, pl.ds(LANES * j, LANES)] = c0[j] * rstd * g[j] + b[j]

            pltpu.sync_copy(rows_v, out_hbm.at[pl.ds(row0, CHUNK)])
            return carry

        lax.fori_loop(0, n_chunks, chunk_fn, 0)

    return sc_embed


def kernel(x, tok_table, pos_table, gamma, beta):
    bsz, seq = x.shape
    n_rows = bsz * seq
    assert n_rows % (NUM_WORKERS * CHUNK) == 0
    assert seq == MAXLEN and tok_table.shape[1] == D_MODEL
    x_flat = x.reshape(n_rows // SUB, SUB).astype(jnp.int32)
    sc = _make_sc_kernel(n_rows)
    out = sc(x_flat, tok_table, pos_table, gamma, beta)
    return out.reshape(bsz, seq, D_MODEL)
